# baseline (speedup 1.0000x reference)
import functools

import jax
import jax.numpy as jnp
from jax import lax
from jax.experimental import pallas as pl
from jax.experimental.pallas import tpu as pltpu

_INF_BITS = 0x7F800000


def _triplet_kernel(emb_ref, labr_ref, labc_ref, mm_ref, out_ref,
                    embn_ref, embt_ref, acc_ref, *, blk_r, n_blk, b, d):
    i = pl.program_id(0)

    @pl.when(i == 0)
    def _init():
        e = emb_ref[...]
        nrm = jnp.sqrt(jnp.sum(e * e, axis=1, keepdims=True))
        en = e / jnp.maximum(nrm, 1e-12)
        embn_ref[...] = en
        embt_ref[...] = en.T
        acc_ref[0] = 0.0
        acc_ref[1] = 0.0

    r0 = i * blk_r
    rows = embn_ref[pl.ds(r0, blk_r), :]
    ent = embt_ref[...]
    g = jnp.dot(rows, ent, preferred_element_type=jnp.float32)
    sq_cols = jnp.sum(ent * ent, axis=0, keepdims=True)
    sq_rows = jnp.sum(rows * rows, axis=1, keepdims=True)
    d2 = sq_rows + sq_cols - 2.0 * g

    lab_cols = labr_ref[...]
    lab_rows = labc_ref[pl.ds(r0, blk_r), :]
    same = lab_rows == lab_cols
    col_ids = lax.broadcasted_iota(jnp.int32, (blk_r, b), 1)
    row_ids = lax.broadcasted_iota(jnp.int32, (blk_r, b), 0) + r0

    neg_inf = jnp.float32(-jnp.inf)
    pos_mask = same & (col_ids != row_ids)
    d2_ap = jnp.max(jnp.where(pos_mask, d2, neg_inf), axis=1, keepdims=True)
    has_pos = d2_ap > neg_inf
    d2_ap_c = jnp.maximum(d2_ap, 0.0)
    d_ap = jnp.sqrt(d2_ap_c)

    n_lab = mm_ref.shape[0]
    oh_rows = (lab_rows == lax.broadcasted_iota(jnp.int32, (1, n_lab), 1)
               ).astype(jnp.float32)
    oh_cols = (lax.broadcasted_iota(jnp.int32, (n_lab, 1), 0) == lab_cols
               ).astype(jnp.float32)
    mrow = jnp.dot(oh_rows, mm_ref[...], preferred_element_type=jnp.float32)
    thr_tab = d_ap + mrow
    thr2_tab = thr_tab * thr_tab
    thr2 = jnp.dot(thr2_tab, oh_cols, preferred_element_type=jnp.float32)

    semi = (~same) & (d2 > d2_ap_c) & (d2 < thr2)

    enc = (lax.bitcast_convert_type(d2, jnp.int32) & jnp.int32(~7)) | lab_cols
    enc_min = jnp.min(jnp.where(semi, enc, jnp.int32(_INF_BITS)),
                      axis=1, keepdims=True)
    has_neg = enc_min < jnp.int32(_INF_BITS)
    lab_n = enc_min & 7
    d2_an = lax.bitcast_convert_type(enc_min & jnp.int32(~7), jnp.float32)
    d_an = jnp.sqrt(d2_an)

    m_sel = jnp.sum(jnp.where(
        lab_n == lax.broadcasted_iota(jnp.int32, (1, n_lab), 1), mrow, 0.0),
        axis=1, keepdims=True)

    valid = has_pos & has_neg
    loss_i = jnp.maximum(d_ap - d_an + m_sel, 0.0)
    contrib = jnp.where(valid, loss_i, 0.0)
    acc_ref[0] = acc_ref[0] + jnp.sum(contrib)
    acc_ref[1] = acc_ref[1] + jnp.sum(jnp.where(valid, 1.0, 0.0))

    @pl.when(i == n_blk - 1)
    def _finish():
        total = acc_ref[0]
        cnt = acc_ref[1]
        out_ref[0, 0] = jnp.where(cnt > 0.0,
                                  total / jnp.maximum(cnt, 1.0),
                                  0.0)


def kernel(embeddings, labels, margin_matrix):
    b, d = embeddings.shape
    blk_r = 256
    n_blk = b // blk_r
    lab_row = labels.reshape(1, b)
    lab_col = labels.reshape(b, 1)
    n_lab = margin_matrix.shape[0]
    out = pl.pallas_call(
        functools.partial(_triplet_kernel, blk_r=blk_r, n_blk=n_blk, b=b, d=d),
        grid=(n_blk,),
        in_specs=[
            pl.BlockSpec((b, d), lambda i: (0, 0)),
            pl.BlockSpec((1, b), lambda i: (0, 0)),
            pl.BlockSpec((b, 1), lambda i: (0, 0)),
            pl.BlockSpec((n_lab, n_lab), lambda i: (0, 0)),
        ],
        out_specs=pl.BlockSpec(memory_space=pltpu.SMEM),
        out_shape=jax.ShapeDtypeStruct((1, 1), jnp.float32),
        scratch_shapes=[
            pltpu.VMEM((b, d), jnp.float32),
            pltpu.VMEM((d, b), jnp.float32),
            pltpu.SMEM((2,), jnp.float32),
        ],
    )(embeddings, lab_row, lab_col, margin_matrix)
    return out[0, 0]


# speedup vs baseline: 1.1373x; 1.1373x over previous
import functools

import jax
import jax.numpy as jnp
from jax import lax
from jax.experimental import pallas as pl
from jax.experimental.pallas import tpu as pltpu

_INF_BITS = 0x7F800000


def _triplet_kernel(emb_ref, labr_ref, labc_ref, mm_ref, out_ref,
                    embn_ref, embt_ref, acc_ref, *, blk_r, n_blk, b, d):
    i = pl.program_id(0)

    @pl.when(i == 0)
    def _init():
        e = emb_ref[...]
        nrm = jnp.sqrt(jnp.sum(e * e, axis=1, keepdims=True))
        en = e / jnp.maximum(nrm, 1e-12)
        embn_ref[...] = en
        embt_ref[...] = en.T
        acc_ref[0] = 0.0
        acc_ref[1] = 0.0

    r0 = i * blk_r
    rows = embn_ref[pl.ds(r0, blk_r), :]
    ent = embt_ref[...]
    g = jnp.dot(rows, ent, preferred_element_type=jnp.float32)
    sq_cols = jnp.sum(ent * ent, axis=0, keepdims=True)
    sq_rows = jnp.sum(rows * rows, axis=1, keepdims=True)
    d2 = sq_rows + sq_cols - 2.0 * g

    lab_cols = labr_ref[...]
    lab_rows = labc_ref[pl.ds(r0, blk_r), :]
    same = lab_rows == lab_cols
    col_ids = lax.broadcasted_iota(jnp.int32, (blk_r, b), 1)
    row_ids = lax.broadcasted_iota(jnp.int32, (blk_r, b), 0) + r0

    neg_inf = jnp.float32(-jnp.inf)
    pos_mask = same & (col_ids != row_ids)
    d2_ap = jnp.max(jnp.where(pos_mask, d2, neg_inf), axis=1, keepdims=True)
    has_pos = d2_ap > neg_inf
    d2_ap_c = jnp.maximum(d2_ap, 0.0)
    d_ap = jnp.sqrt(d2_ap_c)

    n_lab = mm_ref.shape[0]
    oh_rows = (lab_rows == lax.broadcasted_iota(jnp.int32, (1, n_lab), 1)
               ).astype(jnp.float32)
    oh_cols = (lax.broadcasted_iota(jnp.int32, (n_lab, 1), 0) == lab_cols
               ).astype(jnp.float32)
    mrow = jnp.dot(oh_rows, mm_ref[...], preferred_element_type=jnp.float32)
    margins = jnp.dot(mrow, oh_cols, preferred_element_type=jnp.float32)

    thr = d_ap + margins
    semi = (~same) & (d2 > d2_ap_c) & (d2 < thr * thr)

    enc = (lax.bitcast_convert_type(d2, jnp.int32) & jnp.int32(~7)) | lab_cols
    enc_min = jnp.min(jnp.where(semi, enc, jnp.int32(_INF_BITS)),
                      axis=1, keepdims=True)
    has_neg = enc_min < jnp.int32(_INF_BITS)
    lab_n = enc_min & 7
    d2_an = lax.bitcast_convert_type(enc_min & jnp.int32(~7), jnp.float32)
    d_an = jnp.sqrt(d2_an)

    m_sel = jnp.sum(jnp.where(
        lab_n == lax.broadcasted_iota(jnp.int32, (1, n_lab), 1), mrow, 0.0),
        axis=1, keepdims=True)

    valid = has_pos & has_neg
    loss_i = jnp.maximum(d_ap - d_an + m_sel, 0.0)
    contrib = jnp.where(valid, loss_i, 0.0)
    acc_ref[0] = acc_ref[0] + jnp.sum(contrib)
    acc_ref[1] = acc_ref[1] + jnp.sum(jnp.where(valid, 1.0, 0.0))

    @pl.when(i == n_blk - 1)
    def _finish():
        total = acc_ref[0]
        cnt = acc_ref[1]
        out_ref[0, 0] = jnp.where(cnt > 0.0,
                                  total / jnp.maximum(cnt, 1.0),
                                  0.0)


def kernel(embeddings, labels, margin_matrix):
    b, d = embeddings.shape
    blk_r = 512
    n_blk = b // blk_r
    lab_row = labels.reshape(1, b)
    lab_col = labels.reshape(b, 1)
    n_lab = margin_matrix.shape[0]
    out = pl.pallas_call(
        functools.partial(_triplet_kernel, blk_r=blk_r, n_blk=n_blk, b=b, d=d),
        grid=(n_blk,),
        in_specs=[
            pl.BlockSpec((b, d), lambda i: (0, 0)),
            pl.BlockSpec((1, b), lambda i: (0, 0)),
            pl.BlockSpec((b, 1), lambda i: (0, 0)),
            pl.BlockSpec((n_lab, n_lab), lambda i: (0, 0)),
        ],
        out_specs=pl.BlockSpec(memory_space=pltpu.SMEM),
        out_shape=jax.ShapeDtypeStruct((1, 1), jnp.float32),
        scratch_shapes=[
            pltpu.VMEM((b, d), jnp.float32),
            pltpu.VMEM((d, b), jnp.float32),
            pltpu.SMEM((2,), jnp.float32),
        ],
    )(embeddings, lab_row, lab_col, margin_matrix)
    return out[0, 0]
